# Initial kernel scaffold; baseline (speedup 1.0000x reference)
import functools

import jax
import jax.numpy as jnp
import numpy as np
from jax import lax
from jax.experimental import pallas as pl
from jax.experimental.pallas import tpu as pltpu
from jax.experimental.pallas import tpu_sc as plsc

_Z_DIM = 64
_MAX_NORM = float(np.sqrt(_Z_DIM))

_NC = 2
_NS = 16
_NW = _NC * _NS


def _make_sc_gather(batch, traj_len, z_dim):
    rows_per_w = batch // _NW
    mesh = plsc.VectorSubcoreMesh(core_axis_name="c", subcore_axis_name="s")

    @functools.partial(
        pl.kernel,
        mesh=mesh,
        out_type=jax.ShapeDtypeStruct((batch, traj_len, z_dim), jnp.float32),
        scratch_types=[
            pltpu.VMEM((rows_per_w, traj_len), jnp.int32),
            pltpu.VMEM((2, traj_len, z_dim), jnp.float32),
            pltpu.SemaphoreType.DMA,
            pltpu.SemaphoreType.DMA,
        ],
    )
    def sc_gather(noise_hbm, table_hbm, out_hbm, idx_v, rows_v, sem_a, sem_b):
        wid = lax.axis_index("s") * _NC + lax.axis_index("c")
        b0 = wid * rows_per_w
        pltpu.sync_copy(noise_hbm.at[pl.ds(b0, rows_per_w)], idx_v)
        sems = (sem_a, sem_b)
        pltpu.make_async_copy(
            table_hbm.at[idx_v.at[0]], rows_v.at[0], sems[0]).start()
        for j in range(rows_per_w):
            if j + 1 < rows_per_w:
                pltpu.make_async_copy(
                    table_hbm.at[idx_v.at[j + 1]], rows_v.at[(j + 1) % 2],
                    sems[(j + 1) % 2]).start()
            pltpu.make_async_copy(
                table_hbm.at[idx_v.at[j]], rows_v.at[j % 2],
                sems[j % 2]).wait()
            pltpu.sync_copy(rows_v.at[j % 2], out_hbm.at[b0 + j])

    return sc_gather


def _tc_interp_body(nf_ref, t_ref, g_ref, out_ref, *, bb, traj_len):
    nf = nf_ref[0]
    nf_f = nf.astype(jnp.float32)
    g = g_ref[...]
    ss = jnp.sum(g * g, axis=-1, keepdims=True)
    norm = jnp.sqrt(ss)
    scale = jnp.minimum(1.0, _MAX_NORM / jnp.maximum(norm, 1e-12))
    gs = g * scale
    tv = t_ref[...]
    li = tv // nf
    w = (tv % nf).astype(jnp.float32) / nf_f
    f_cnt = tv.shape[1]
    j = lax.broadcasted_iota(jnp.int32, (traj_len, f_cnt), 0)
    for b in range(bb):
        li_b = li[b:b + 1, :]
        w_b = w[b:b + 1, :]
        wt = (jnp.where(j == li_b, 1.0 - w_b, 0.0)
              + jnp.where(j == li_b + 1, w_b, 0.0))
        out_ref[b] = lax.dot_general(
            wt, gs[b], (((0,), (0,)), ((), ())),
            preferred_element_type=jnp.float32,
            precision=lax.Precision.HIGHEST)


def _make_tc_interp(batch, f_cnt, traj_len, z_dim, bb):
    grid = (batch // bb,)
    return pl.pallas_call(
        functools.partial(_tc_interp_body, bb=bb, traj_len=traj_len),
        grid=grid,
        in_specs=[
            pl.BlockSpec(memory_space=pltpu.SMEM),
            pl.BlockSpec((bb, f_cnt), lambda b: (b, 0)),
            pl.BlockSpec((bb, traj_len, z_dim), lambda b: (b, 0, 0)),
        ],
        out_specs=pl.BlockSpec((bb, f_cnt, z_dim), lambda b: (b, 0, 0)),
        out_shape=jax.ShapeDtypeStruct((batch, f_cnt, z_dim), jnp.float32),
    )


def kernel(c, t, l, num_frames_per_motion, motion_noise, embed_weight):
    batch, f_cnt = t.shape
    traj_len = motion_noise.shape[1]
    z_dim = embed_weight.shape[1]
    noise = motion_noise.astype(jnp.int32)
    gathered = _make_sc_gather(batch, traj_len, z_dim)(noise, embed_weight)
    nf = jnp.asarray(num_frames_per_motion, jnp.int32).reshape(1)
    bb = 8
    return _make_tc_interp(batch, f_cnt, traj_len, z_dim, bb)(
        nf, t.astype(jnp.int32), gathered)


# speedup vs baseline: 17.6734x; 17.6734x over previous
import functools

import jax
import jax.numpy as jnp
import numpy as np
from jax import lax
from jax.experimental import pallas as pl
from jax.experimental.pallas import tpu as pltpu
from jax.experimental.pallas import tpu_sc as plsc

_Z_DIM = 64
_MAX_NORM = float(np.sqrt(_Z_DIM))

_NC = 2
_NS = 16
_NW = _NC * _NS


def _make_sc_gather(batch, traj_len, z_dim):
    rows_per_w = batch // _NW
    mesh = plsc.VectorSubcoreMesh(core_axis_name="c", subcore_axis_name="s")

    @functools.partial(
        pl.kernel,
        mesh=mesh,
        compiler_params=pltpu.CompilerParams(use_tc_tiling_on_sc=False),
        out_type=jax.ShapeDtypeStruct((batch, traj_len, z_dim), jnp.float32),
        scratch_types=[
            pltpu.VMEM((rows_per_w, traj_len), jnp.int32),
            pltpu.VMEM((2, traj_len, z_dim), jnp.float32),
            pltpu.SemaphoreType.DMA,
            pltpu.SemaphoreType.DMA,
        ],
    )
    def sc_gather(noise_hbm, table_hbm, out_hbm, idx_v, rows_v, sem_a, sem_b):
        wid = lax.axis_index("s") * _NC + lax.axis_index("c")
        b0 = wid * rows_per_w
        pltpu.sync_copy(noise_hbm.at[pl.ds(b0, rows_per_w)], idx_v)
        sems = (sem_a, sem_b)
        pltpu.make_async_copy(
            table_hbm.at[idx_v.at[0]], rows_v.at[0], sems[0]).start()
        for j in range(rows_per_w):
            if j + 1 < rows_per_w:
                pltpu.make_async_copy(
                    table_hbm.at[idx_v.at[j + 1]], rows_v.at[(j + 1) % 2],
                    sems[(j + 1) % 2]).start()
            pltpu.make_async_copy(
                table_hbm.at[idx_v.at[j]], rows_v.at[j % 2],
                sems[j % 2]).wait()
            pltpu.sync_copy(rows_v.at[j % 2], out_hbm.at[b0 + j])

    return sc_gather


def _tc_interp_body(nf_ref, t_ref, g_ref, out_ref, *, bb, traj_len):
    nf = nf_ref[0]
    nf_f = nf.astype(jnp.float32)
    g = g_ref[...]
    ss = jnp.sum(g * g, axis=-1, keepdims=True)
    scale = jnp.minimum(
        1.0, _MAX_NORM * lax.rsqrt(jnp.maximum(ss, 1e-24)))
    gs = (g * scale).astype(jnp.bfloat16)
    tv = t_ref[...]
    tf = tv.astype(jnp.float32) * (1.0 / nf_f)
    f_cnt = tv.shape[1]
    j = lax.broadcasted_iota(
        jnp.int32, (traj_len, f_cnt), 0).astype(jnp.float32)
    for b in range(bb):
        d = (j - tf[b:b + 1, :]).astype(jnp.bfloat16)
        wt = jnp.maximum(1.0 - jnp.abs(d), 0.0).astype(jnp.bfloat16)
        out_ref[b] = lax.dot_general(
            wt, gs[b], (((0,), (0,)), ((), ())),
            preferred_element_type=jnp.float32)


def _make_tc_interp(batch, f_cnt, traj_len, z_dim, bb):
    grid = (batch // bb,)
    return pl.pallas_call(
        functools.partial(_tc_interp_body, bb=bb, traj_len=traj_len),
        grid=grid,
        in_specs=[
            pl.BlockSpec(memory_space=pltpu.SMEM),
            pl.BlockSpec((bb, f_cnt), lambda b: (b, 0)),
            pl.BlockSpec((bb, traj_len, z_dim), lambda b: (b, 0, 0)),
        ],
        out_specs=pl.BlockSpec((bb, f_cnt, z_dim), lambda b: (b, 0, 0)),
        out_shape=jax.ShapeDtypeStruct((batch, f_cnt, z_dim), jnp.float32),
    )


def kernel(c, t, l, num_frames_per_motion, motion_noise, embed_weight):
    batch, f_cnt = t.shape
    traj_len = motion_noise.shape[1]
    z_dim = embed_weight.shape[1]
    noise = motion_noise.astype(jnp.int32)
    gathered = _make_sc_gather(batch, traj_len, z_dim)(noise, embed_weight)
    nf = jnp.asarray(num_frames_per_motion, jnp.int32).reshape(1)
    bb = 8
    return _make_tc_interp(batch, f_cnt, traj_len, z_dim, bb)(
        nf, t.astype(jnp.int32), gathered)
